# baseline (speedup 1.0000x reference)
import functools

import jax
import jax.numpy as jnp
from jax import lax
from jax.experimental import pallas as pl
from jax.experimental.pallas import tpu as pltpu
from jax.experimental.pallas import tpu_sc as plsc

_NC = 2
_NS = 16
_L = 16


def _make_sc_selector(bs, n):
    bpc = bs // _NC
    qrt = _NS // bpc
    slc = n // qrt
    nvec0 = slc // _L

    mesh = plsc.VectorSubcoreMesh(
        core_axis_name="c", subcore_axis_name="s",
        num_cores=_NC, num_subcores=_NS)

    @functools.partial(
        pl.kernel,
        out_type=jax.ShapeDtypeStruct((bs, qrt * _L), jnp.float32),
        mesh=mesh,
        compiler_params=pltpu.CompilerParams(needs_layout_passes=False),
        scratch_types=[
            pltpu.VMEM((slc + 8 * _L,), jnp.int32),
            pltpu.VMEM((256 * _L,), jnp.int32),
            pltpu.VMEM((qrt, 256), jnp.int32),
            pltpu.VMEM((_L,), jnp.int32),
            pltpu.VMEM((_L,), jnp.float32),
            pltpu.VMEM_SHARED((bpc, qrt, 256), jnp.int32),
        ],
    )
    def selector(x_hbm, params_hbm, out_hbm, buf, hist, hist4, pv, qv,
                 sh_hist):
        INT_MIN = jnp.int32(-(2**31))
        INT_MAX = jnp.int32(2**31 - 1)
        LOW31 = jnp.int32(0x7FFFFFFF)
        iota = lax.broadcasted_iota(jnp.int32, (_L,), 0)
        zeros16 = jnp.zeros((_L,), jnp.int32)
        ones16 = jnp.ones((_L,), jnp.int32)

        c = lax.axis_index("c")
        s = lax.axis_index("s")
        bl = s // qrt
        qt = s % qrt
        batch = c * bpc + bl

        pltpu.sync_copy(params_hbm, pv)
        pvec = pv[...]
        k_lo = jnp.sum(jnp.where(iota == 0, pvec, 0))
        k_hi = jnp.sum(jnp.where(iota == 1, pvec, 0))
        frac = lax.bitcast_convert_type(
            jnp.sum(jnp.where(iota == 2, pvec, 0)), jnp.float32)

        pltpu.sync_copy(x_hbm.at[batch, pl.ds(qt * slc, slc)],
                        buf.at[pl.ds(0, slc)])

        def zero_hist():
            def zbody(j, _):
                for u in range(8):
                    hist[pl.ds((j * 8 + u) * _L, _L)] = zeros16
                return 0
            lax.fori_loop(0, 256 // 8, zbody, 0)

        def hist_sweep(nvec, active, shift, transform):
            lanebase = iota * 256

            def body(i, _):
                for u in range(8):
                    off = (i * 8 + u) * _L
                    v = buf[pl.ds(off, _L)]
                    if transform:
                        v = jnp.where(v < 0, v ^ LOW31, v)
                        buf[pl.ds(off, _L)] = v
                    valid = (off + iota) < active
                    digit = ((v ^ INT_MIN) >> shift) & 255
                    plsc.addupdate_scatter(hist, [lanebase + digit], ones16,
                                           mask=valid)
                return 0
            lax.fori_loop(0, (nvec + 7) // 8, body, 0)

        def publish_hist():
            for j in range(256 // _L):
                acc = zeros16
                for l in range(_L):
                    acc = acc + hist[pl.ds(l * 256 + j * _L, _L)]
                hist4[qt, pl.ds(j * _L, _L)] = acc
            pltpu.sync_copy(hist4.at[qt], sh_hist.at[bl, qt])

        def walk(rem):
            pltpu.sync_copy(sh_hist.at[bl], hist4)
            total = zeros16
            dstar = zeros16
            below = zeros16
            cnt_d = zeros16
            found = zeros16
            for j in range(256 // _L):
                g = zeros16
                for qq in range(qrt):
                    g = g + hist4[qq, pl.ds(j * _L, _L)]
                cum = plsc.cumsum(g)
                hit = (total + cum) > rem
                npc = plsc.all_reduce_population_count(hit)
                anyhit = jnp.where(npc > 0, ones16, zeros16)
                ffs = plsc.all_reduce_ffs(hit)
                newly = anyhit * (1 - found)
                dj = jnp.full((_L,), j * _L, jnp.int32) + ffs
                bj = total + jnp.full(
                    (_L,), jnp.sum(jnp.where(iota < ffs, g, 0)), jnp.int32)
                cj = jnp.full((_L,), jnp.sum(jnp.where(iota == ffs, g, 0)),
                              jnp.int32)
                dstar = jnp.where(newly > 0, dj, dstar)
                below = jnp.where(newly > 0, bj, below)
                cnt_d = jnp.where(newly > 0, cj, cnt_d)
                found = jnp.maximum(found, anyhit)
                total = total + jnp.full((_L,), jnp.sum(g), jnp.int32)
            return dstar, below, cnt_d

        def compact_sweep(nvec, active, shift, dstar):
            def body(i, carry):
                wp, mina = carry
                for u in range(8):
                    off = (i * 8 + u) * _L
                    v = buf[pl.ds(off, _L)]
                    valid = (off + iota) < active
                    digit = ((v ^ INT_MIN) >> shift) & 255
                    keep = valid & (digit == dstar)
                    above = valid & (digit > dstar)
                    mina = jnp.where(above, jnp.minimum(mina, v), mina)
                    plsc.store_compressed(buf.at[pl.ds(wp, _L)], v, mask=keep)
                    npc = plsc.all_reduce_population_count(keep)
                    wp = wp + jnp.sum(jnp.where(iota == 0, npc, 0))
                return wp, mina
            wp, mina = lax.fori_loop(
                0, (nvec + 7) // 8, body,
                (jnp.int32(0), jnp.full((_L,), INT_MAX, jnp.int32)))
            return wp, mina

        rem = jnp.full((_L,), k_lo, jnp.int32)
        ukey = zeros16
        le_below = zeros16
        succ = jnp.full((_L,), INT_MAX, jnp.int32)
        active = jnp.int32(slc)
        cnt_last = zeros16
        for lvl in range(4):
            shift = jnp.int32(24 - 8 * lvl)
            zero_hist()
            nvec = nvec0 if lvl == 0 else (active + (_L - 1)) // _L
            hist_sweep(nvec, active, shift, transform=(lvl == 0))
            publish_hist()
            plsc.subcore_barrier()
            dstar, below, cnt_d = walk(rem)
            plsc.subcore_barrier()
            wp, mina = compact_sweep(nvec, active, shift, dstar)
            ukey = ukey + (dstar << shift)
            rem = rem - below
            le_below = le_below + below
            succ = jnp.minimum(succ, mina)
            active = wp
            cnt_last = cnt_d

        key_lo = ukey ^ INT_MIN
        hist4[qt, pl.ds(0, _L)] = succ
        pltpu.sync_copy(hist4.at[qt], sh_hist.at[bl, qt])
        plsc.subcore_barrier()
        pltpu.sync_copy(sh_hist.at[bl], hist4)
        gsucc = jnp.full((_L,), INT_MAX, jnp.int32)
        for qq in range(qrt):
            gsucc = jnp.minimum(gsucc, hist4[qq, pl.ds(0, _L)])
        gsucc = jnp.full((_L,), jnp.min(gsucc), jnp.int32)
        le = le_below + cnt_last
        key_hi = jnp.where(le > k_hi, key_lo, gsucc)

        def tofloat(a):
            bits = jnp.where(a < 0, a ^ LOW31, a)
            return lax.bitcast_convert_type(bits, jnp.float32)

        v_lo = tofloat(key_lo)
        v_hi = tofloat(key_hi)
        fr = jnp.full((_L,), frac, jnp.float32)
        q = v_lo * (jnp.float32(1.0) - fr) + v_hi * fr
        qv[...] = q
        pltpu.sync_copy(qv, out_hbm.at[batch, pl.ds(qt * _L, _L)])

    return selector


def _mask_kernel(q_ref, x_ref, out_ref):
    out_ref[...] = (x_ref[...] >= q_ref[...]).astype(jnp.float32)


def kernel(scale, pr):
    bs, ch, w, h = scale.shape
    n = ch * w * h
    flat = scale.reshape(bs, n)

    pr_arr = jnp.asarray(pr, jnp.int32)
    pr_f = jnp.where(pr_arr > 10, 10, pr_arr) * jnp.float32(0.1)
    pr_bis = jnp.float32(1.0) - pr_f
    idx = pr_bis * jnp.float32(n - 1)
    low = jnp.floor(idx)
    frac = jnp.clip(idx - low, 0.0, 1.0)
    k_lo = jnp.clip(low.astype(jnp.int32), 0, n - 1)
    k_hi = jnp.clip(jnp.ceil(idx).astype(jnp.int32), 0, n - 1)

    params = jnp.zeros((16,), jnp.int32)
    params = params.at[0].set(k_lo).at[1].set(k_hi)
    params = params.at[2].set(lax.bitcast_convert_type(frac, jnp.int32))

    xi32 = lax.bitcast_convert_type(flat, jnp.int32)
    selector = _make_sc_selector(bs, n)
    q2 = selector(xi32, params)
    q8 = q2[:, :1]
    q8 = jnp.where(pr_arr == 0, jnp.float32(jnp.inf), q8)
    q8 = jnp.where(pr_arr >= 10, jnp.float32(-jnp.inf), q8)

    out = pl.pallas_call(
        _mask_kernel,
        out_shape=jax.ShapeDtypeStruct((bs, n), jnp.float32),
        in_specs=[
            pl.BlockSpec(memory_space=pltpu.VMEM),
            pl.BlockSpec(memory_space=pltpu.VMEM),
        ],
        out_specs=pl.BlockSpec(memory_space=pltpu.VMEM),
    )(q8, flat)
    return out.reshape(bs, ch, w, h)


# speedup vs baseline: 1.0651x; 1.0651x over previous
import functools

import jax
import jax.numpy as jnp
from jax import lax
from jax.experimental import pallas as pl
from jax.experimental.pallas import tpu as pltpu
from jax.experimental.pallas import tpu_sc as plsc

_NC = 2
_NS = 16
_L = 16


def _make_sc_selector(bs, n):
    bpc = bs // _NC
    qrt = _NS // bpc
    slc = n // qrt
    nvec0 = slc // _L

    mesh = plsc.VectorSubcoreMesh(
        core_axis_name="c", subcore_axis_name="s",
        num_cores=_NC, num_subcores=_NS)

    @functools.partial(
        pl.kernel,
        out_type=jax.ShapeDtypeStruct((bs, qrt * _L), jnp.float32),
        mesh=mesh,
        compiler_params=pltpu.CompilerParams(needs_layout_passes=False),
        scratch_types=[
            pltpu.VMEM((slc + 8 * _L,), jnp.int32),
            pltpu.VMEM((256 * _L,), jnp.int32),
            pltpu.VMEM((qrt, 256), jnp.int32),
            pltpu.VMEM((_L,), jnp.int32),
            pltpu.VMEM((_L,), jnp.float32),
            pltpu.VMEM_SHARED((bpc, qrt, 256), jnp.int32),
        ],
    )
    def selector(x_hbm, params_hbm, out_hbm, buf, hist, hist4, pv, qv,
                 sh_hist):
        INT_MIN = jnp.int32(-(2**31))
        INT_MAX = jnp.int32(2**31 - 1)
        LOW31 = jnp.int32(0x7FFFFFFF)
        iota = lax.broadcasted_iota(jnp.int32, (_L,), 0)
        zeros16 = jnp.zeros((_L,), jnp.int32)
        ones16 = jnp.ones((_L,), jnp.int32)

        c = lax.axis_index("c")
        s = lax.axis_index("s")
        bl = s // qrt
        qt = s % qrt
        batch = c * bpc + bl

        pltpu.sync_copy(params_hbm, pv)
        pvec = pv[...]
        k_lo = jnp.sum(jnp.where(iota == 0, pvec, 0))
        k_hi = jnp.sum(jnp.where(iota == 1, pvec, 0))
        frac = lax.bitcast_convert_type(
            jnp.sum(jnp.where(iota == 2, pvec, 0)), jnp.float32)

        pltpu.sync_copy(x_hbm.at[batch, pl.ds(qt * slc, slc)],
                        buf.at[pl.ds(0, slc)])

        def zero_hist():
            def zbody(j, _):
                for u in range(8):
                    hist[pl.ds((j * 8 + u) * _L, _L)] = zeros16
                return 0
            lax.fori_loop(0, 256 // 8, zbody, 0)

        def hist_sweep(nvec, active, shift, transform):
            def body(i, _):
                for u in range(8):
                    off = (i * 8 + u) * _L
                    v = buf[pl.ds(off, _L)]
                    if transform:
                        v = jnp.where(v < 0, v ^ LOW31, v)
                        buf[pl.ds(off, _L)] = v
                    valid = (off + iota) < active
                    digit = ((v ^ INT_MIN) >> shift) & 255
                    plsc.addupdate_scatter(hist, [digit * _L + iota], ones16,
                                           mask=valid)
                return 0
            lax.fori_loop(0, (nvec + 7) // 8, body, 0)

        def publish_hist():
            for j in range(256 // _L):
                acc = zeros16
                for t in range(_L):
                    s = jnp.sum(hist[pl.ds((j * _L + t) * _L, _L)])
                    acc = acc + jnp.where(iota == t, s, 0)
                hist4[qt, pl.ds(j * _L, _L)] = acc
            pltpu.sync_copy(hist4.at[qt], sh_hist.at[bl, qt])

        def walk(rem):
            pltpu.sync_copy(sh_hist.at[bl], hist4)
            total = zeros16
            dstar = zeros16
            below = zeros16
            cnt_d = zeros16
            found = zeros16
            for j in range(256 // _L):
                g = zeros16
                for qq in range(qrt):
                    g = g + hist4[qq, pl.ds(j * _L, _L)]
                cum = plsc.cumsum(g)
                hit = (total + cum) > rem
                npc = plsc.all_reduce_population_count(hit)
                anyhit = jnp.where(npc > 0, ones16, zeros16)
                ffs = plsc.all_reduce_ffs(hit)
                newly = anyhit * (1 - found)
                dj = jnp.full((_L,), j * _L, jnp.int32) + ffs
                bj = total + jnp.full(
                    (_L,), jnp.sum(jnp.where(iota < ffs, g, 0)), jnp.int32)
                cj = jnp.full((_L,), jnp.sum(jnp.where(iota == ffs, g, 0)),
                              jnp.int32)
                dstar = jnp.where(newly > 0, dj, dstar)
                below = jnp.where(newly > 0, bj, below)
                cnt_d = jnp.where(newly > 0, cj, cnt_d)
                found = jnp.maximum(found, anyhit)
                total = total + jnp.full((_L,), jnp.sum(g), jnp.int32)
            return dstar, below, cnt_d

        def compact_sweep(nvec, active, shift, dstar):
            def body(i, carry):
                wp, mina = carry
                for u in range(8):
                    off = (i * 8 + u) * _L
                    v = buf[pl.ds(off, _L)]
                    valid = (off + iota) < active
                    digit = ((v ^ INT_MIN) >> shift) & 255
                    keep = valid & (digit == dstar)
                    above = valid & (digit > dstar)
                    mina = jnp.where(above, jnp.minimum(mina, v), mina)
                    plsc.store_compressed(buf.at[pl.ds(wp, _L)], v, mask=keep)
                    npc = plsc.all_reduce_population_count(keep)
                    wp = wp + jnp.sum(jnp.where(iota == 0, npc, 0))
                return wp, mina
            wp, mina = lax.fori_loop(
                0, (nvec + 7) // 8, body,
                (jnp.int32(0), jnp.full((_L,), INT_MAX, jnp.int32)))
            return wp, mina

        rem = jnp.full((_L,), k_lo, jnp.int32)
        ukey = zeros16
        le_below = zeros16
        succ = jnp.full((_L,), INT_MAX, jnp.int32)
        active = jnp.int32(slc)
        cnt_last = zeros16
        for lvl in range(4):
            shift = jnp.int32(24 - 8 * lvl)
            zero_hist()
            nvec = nvec0 if lvl == 0 else (active + (_L - 1)) // _L
            hist_sweep(nvec, active, shift, transform=(lvl == 0))
            publish_hist()
            plsc.subcore_barrier()
            dstar, below, cnt_d = walk(rem)
            plsc.subcore_barrier()
            wp, mina = compact_sweep(nvec, active, shift, dstar)
            ukey = ukey + (dstar << shift)
            rem = rem - below
            le_below = le_below + below
            succ = jnp.minimum(succ, mina)
            active = wp
            cnt_last = cnt_d

        key_lo = ukey ^ INT_MIN
        hist4[qt, pl.ds(0, _L)] = succ
        pltpu.sync_copy(hist4.at[qt], sh_hist.at[bl, qt])
        plsc.subcore_barrier()
        pltpu.sync_copy(sh_hist.at[bl], hist4)
        gsucc = jnp.full((_L,), INT_MAX, jnp.int32)
        for qq in range(qrt):
            gsucc = jnp.minimum(gsucc, hist4[qq, pl.ds(0, _L)])
        gsucc = jnp.full((_L,), jnp.min(gsucc), jnp.int32)
        le = le_below + cnt_last
        key_hi = jnp.where(le > k_hi, key_lo, gsucc)

        def tofloat(a):
            bits = jnp.where(a < 0, a ^ LOW31, a)
            return lax.bitcast_convert_type(bits, jnp.float32)

        v_lo = tofloat(key_lo)
        v_hi = tofloat(key_hi)
        fr = jnp.full((_L,), frac, jnp.float32)
        q = v_lo * (jnp.float32(1.0) - fr) + v_hi * fr
        qv[...] = q
        pltpu.sync_copy(qv, out_hbm.at[batch, pl.ds(qt * _L, _L)])

    return selector


def _mask_kernel(q_ref, x_ref, out_ref):
    out_ref[...] = (x_ref[...] >= q_ref[...]).astype(jnp.float32)


def kernel(scale, pr):
    bs, ch, w, h = scale.shape
    n = ch * w * h
    flat = scale.reshape(bs, n)

    pr_arr = jnp.asarray(pr, jnp.int32)
    pr_f = jnp.where(pr_arr > 10, 10, pr_arr) * jnp.float32(0.1)
    pr_bis = jnp.float32(1.0) - pr_f
    idx = pr_bis * jnp.float32(n - 1)
    low = jnp.floor(idx)
    frac = jnp.clip(idx - low, 0.0, 1.0)
    k_lo = jnp.clip(low.astype(jnp.int32), 0, n - 1)
    k_hi = jnp.clip(jnp.ceil(idx).astype(jnp.int32), 0, n - 1)

    params = jnp.zeros((16,), jnp.int32)
    params = params.at[0].set(k_lo).at[1].set(k_hi)
    params = params.at[2].set(lax.bitcast_convert_type(frac, jnp.int32))

    xi32 = lax.bitcast_convert_type(flat, jnp.int32)
    selector = _make_sc_selector(bs, n)
    q2 = selector(xi32, params)
    q8 = q2[:, :1]
    q8 = jnp.where(pr_arr == 0, jnp.float32(jnp.inf), q8)
    q8 = jnp.where(pr_arr >= 10, jnp.float32(-jnp.inf), q8)

    out = pl.pallas_call(
        _mask_kernel,
        out_shape=jax.ShapeDtypeStruct((bs, n), jnp.float32),
        in_specs=[
            pl.BlockSpec(memory_space=pltpu.VMEM),
            pl.BlockSpec(memory_space=pltpu.VMEM),
        ],
        out_specs=pl.BlockSpec(memory_space=pltpu.VMEM),
    )(q8, flat)
    return out.reshape(bs, ch, w, h)


# speedup vs baseline: 1.0969x; 1.0298x over previous
import functools

import jax
import jax.numpy as jnp
from jax import lax
from jax.experimental import pallas as pl
from jax.experimental.pallas import tpu as pltpu
from jax.experimental.pallas import tpu_sc as plsc

_NC = 2
_NS = 16
_L = 16


def _make_sc_selector(bs, n):
    bpc = bs // _NC
    qrt = _NS // bpc
    slc = n // qrt
    nvec0 = slc // _L

    mesh = plsc.VectorSubcoreMesh(
        core_axis_name="c", subcore_axis_name="s",
        num_cores=_NC, num_subcores=_NS)

    @functools.partial(
        pl.kernel,
        out_type=jax.ShapeDtypeStruct((bs, qrt * _L), jnp.float32),
        mesh=mesh,
        compiler_params=pltpu.CompilerParams(needs_layout_passes=False),
        scratch_types=[
            pltpu.VMEM((slc + 8 * _L,), jnp.int32),
            pltpu.VMEM((256 * _L,), jnp.int32),
            pltpu.VMEM((qrt, 256), jnp.int32),
            pltpu.VMEM((_L,), jnp.int32),
            pltpu.VMEM((_L,), jnp.float32),
            pltpu.VMEM_SHARED((bpc, qrt, 256), jnp.int32),
        ],
    )
    def selector(x_hbm, params_hbm, out_hbm, buf, hist, hist4, pv, qv,
                 sh_hist):
        INT_MIN = jnp.int32(-(2**31))
        INT_MAX = jnp.int32(2**31 - 1)
        LOW31 = jnp.int32(0x7FFFFFFF)
        iota = lax.broadcasted_iota(jnp.int32, (_L,), 0)
        zeros16 = jnp.zeros((_L,), jnp.int32)
        ones16 = jnp.ones((_L,), jnp.int32)

        c = lax.axis_index("c")
        s = lax.axis_index("s")
        bl = s // qrt
        qt = s % qrt
        batch = c * bpc + bl

        pltpu.sync_copy(params_hbm, pv)
        pvec = pv[...]
        k_lo = jnp.sum(jnp.where(iota == 0, pvec, 0))
        k_hi = jnp.sum(jnp.where(iota == 1, pvec, 0))
        frac = lax.bitcast_convert_type(
            jnp.sum(jnp.where(iota == 2, pvec, 0)), jnp.float32)

        pltpu.sync_copy(x_hbm.at[batch, pl.ds(qt * slc, slc)],
                        buf.at[pl.ds(0, slc)])

        def zero_hist():
            def zbody(j, _):
                for u in range(8):
                    hist[pl.ds((j * 8 + u) * _L, _L)] = zeros16
                return 0
            lax.fori_loop(0, 256 // 8, zbody, 0)

        def hist_sweep(nvec, active, shift, transform, pfx=None):
            def body(i, carry):
                mina = carry
                for u in range(8):
                    off = (i * 8 + u) * _L
                    v = buf[pl.ds(off, _L)]
                    if transform:
                        v = jnp.where(v < 0, v ^ LOW31, v)
                        buf[pl.ds(off, _L)] = v
                    valid = (off + iota) < active
                    uk = v ^ INT_MIN
                    digit = (uk >> shift) & 255
                    if pfx is not None:
                        d_prev = (uk >> (shift + 8)) & 255
                        above = valid & (d_prev > pfx)
                        mina = jnp.where(above, jnp.minimum(mina, v), mina)
                        valid = valid & (d_prev == pfx)
                    plsc.addupdate_scatter(hist, [digit * _L + iota], ones16,
                                           mask=valid)
                return mina
            return lax.fori_loop(0, (nvec + 7) // 8, body,
                                 jnp.full((_L,), INT_MAX, jnp.int32))

        def publish_hist():
            for j in range(256 // _L):
                acc = zeros16
                for t in range(_L):
                    s = jnp.sum(hist[pl.ds((j * _L + t) * _L, _L)])
                    acc = acc + jnp.where(iota == t, s, 0)
                hist4[qt, pl.ds(j * _L, _L)] = acc
            pltpu.sync_copy(hist4.at[qt], sh_hist.at[bl, qt])

        def walk(rem):
            pltpu.sync_copy(sh_hist.at[bl], hist4)
            total = zeros16
            dstar = zeros16
            below = zeros16
            cnt_d = zeros16
            found = zeros16
            for j in range(256 // _L):
                g = zeros16
                for qq in range(qrt):
                    g = g + hist4[qq, pl.ds(j * _L, _L)]
                cum = plsc.cumsum(g)
                hit = (total + cum) > rem
                npc = plsc.all_reduce_population_count(hit)
                anyhit = jnp.where(npc > 0, ones16, zeros16)
                ffs = plsc.all_reduce_ffs(hit)
                newly = anyhit * (1 - found)
                dj = jnp.full((_L,), j * _L, jnp.int32) + ffs
                bj = total + jnp.full(
                    (_L,), jnp.sum(jnp.where(iota < ffs, g, 0)), jnp.int32)
                cj = jnp.full((_L,), jnp.sum(jnp.where(iota == ffs, g, 0)),
                              jnp.int32)
                dstar = jnp.where(newly > 0, dj, dstar)
                below = jnp.where(newly > 0, bj, below)
                cnt_d = jnp.where(newly > 0, cj, cnt_d)
                found = jnp.maximum(found, anyhit)
                total = total + jnp.full((_L,), jnp.sum(g), jnp.int32)
            return dstar, below, cnt_d

        def compact_sweep(nvec, active, shift, dstar, pfx=None):
            def body(i, carry):
                wp, mina = carry
                for u in range(8):
                    off = (i * 8 + u) * _L
                    v = buf[pl.ds(off, _L)]
                    valid = (off + iota) < active
                    uk = v ^ INT_MIN
                    digit = (uk >> shift) & 255
                    if pfx is not None:
                        valid = valid & (((uk >> (shift + 8)) & 255) == pfx)
                    keep = valid & (digit == dstar)
                    above = valid & (digit > dstar)
                    mina = jnp.where(above, jnp.minimum(mina, v), mina)
                    plsc.store_compressed(buf.at[pl.ds(wp, _L)], v, mask=keep)
                    npc = plsc.all_reduce_population_count(keep)
                    wp = wp + jnp.sum(jnp.where(iota == 0, npc, 0))
                return wp, mina
            wp, mina = lax.fori_loop(
                0, (nvec + 7) // 8, body,
                (jnp.int32(0), jnp.full((_L,), INT_MAX, jnp.int32)))
            return wp, mina

        rem = jnp.full((_L,), k_lo, jnp.int32)
        ukey = zeros16
        le_below = zeros16
        succ = jnp.full((_L,), INT_MAX, jnp.int32)
        active = jnp.int32(slc)
        cnt_last = zeros16
        d0star = zeros16
        for lvl in range(4):
            shift = jnp.int32(24 - 8 * lvl)
            zero_hist()
            nvec = nvec0 if lvl <= 1 else (active + (_L - 1)) // _L
            mina_prev = hist_sweep(nvec, active, shift,
                                   transform=(lvl == 0),
                                   pfx=d0star if lvl == 1 else None)
            publish_hist()
            plsc.subcore_barrier()
            dstar, below, cnt_d = walk(rem)
            plsc.subcore_barrier()
            if lvl == 0:
                d0star = dstar
                wp, mina = active, jnp.full((_L,), INT_MAX, jnp.int32)
            else:
                wp, mina = compact_sweep(nvec, active, shift, dstar,
                                         pfx=d0star if lvl == 1 else None)
            ukey = ukey + (dstar << shift)
            rem = rem - below
            le_below = le_below + below
            succ = jnp.minimum(succ, jnp.minimum(mina, mina_prev))
            active = wp
            cnt_last = cnt_d

        key_lo = ukey ^ INT_MIN
        hist4[qt, pl.ds(0, _L)] = succ
        pltpu.sync_copy(hist4.at[qt], sh_hist.at[bl, qt])
        plsc.subcore_barrier()
        pltpu.sync_copy(sh_hist.at[bl], hist4)
        gsucc = jnp.full((_L,), INT_MAX, jnp.int32)
        for qq in range(qrt):
            gsucc = jnp.minimum(gsucc, hist4[qq, pl.ds(0, _L)])
        gsucc = jnp.full((_L,), jnp.min(gsucc), jnp.int32)
        le = le_below + cnt_last
        key_hi = jnp.where(le > k_hi, key_lo, gsucc)

        def tofloat(a):
            bits = jnp.where(a < 0, a ^ LOW31, a)
            return lax.bitcast_convert_type(bits, jnp.float32)

        v_lo = tofloat(key_lo)
        v_hi = tofloat(key_hi)
        fr = jnp.full((_L,), frac, jnp.float32)
        q = v_lo * (jnp.float32(1.0) - fr) + v_hi * fr
        qv[...] = q
        pltpu.sync_copy(qv, out_hbm.at[batch, pl.ds(qt * _L, _L)])

    return selector


def _mask_kernel(q_ref, x_ref, out_ref):
    out_ref[...] = (x_ref[...] >= q_ref[...]).astype(jnp.float32)


def kernel(scale, pr):
    bs, ch, w, h = scale.shape
    n = ch * w * h
    flat = scale.reshape(bs, n)

    pr_arr = jnp.asarray(pr, jnp.int32)
    pr_f = jnp.where(pr_arr > 10, 10, pr_arr) * jnp.float32(0.1)
    pr_bis = jnp.float32(1.0) - pr_f
    idx = pr_bis * jnp.float32(n - 1)
    low = jnp.floor(idx)
    frac = jnp.clip(idx - low, 0.0, 1.0)
    k_lo = jnp.clip(low.astype(jnp.int32), 0, n - 1)
    k_hi = jnp.clip(jnp.ceil(idx).astype(jnp.int32), 0, n - 1)

    params = jnp.zeros((16,), jnp.int32)
    params = params.at[0].set(k_lo).at[1].set(k_hi)
    params = params.at[2].set(lax.bitcast_convert_type(frac, jnp.int32))

    xi32 = lax.bitcast_convert_type(flat, jnp.int32)
    selector = _make_sc_selector(bs, n)
    q2 = selector(xi32, params)
    q8 = q2[:, :1]
    q8 = jnp.where(pr_arr == 0, jnp.float32(jnp.inf), q8)
    q8 = jnp.where(pr_arr >= 10, jnp.float32(-jnp.inf), q8)

    out = pl.pallas_call(
        _mask_kernel,
        out_shape=jax.ShapeDtypeStruct((bs, n), jnp.float32),
        in_specs=[
            pl.BlockSpec(memory_space=pltpu.VMEM),
            pl.BlockSpec(memory_space=pltpu.VMEM),
        ],
        out_specs=pl.BlockSpec(memory_space=pltpu.VMEM),
    )(q8, flat)
    return out.reshape(bs, ch, w, h)
